# baseline (speedup 1.0000x reference)
import functools

import jax
import jax.numpy as jnp
from jax import lax
from jax.experimental import pallas as pl
from jax.experimental.pallas import tpu as pltpu
from jax.experimental.pallas import tpu_sc as plsc

D, H, E, K = 768, 1024, 8, 2
T = 4096
P = T * K
BLK = 128
NBLK = P // BLK
NT = 72
XS = NT * BLK
NC, NS = 2, 16
NW = NC * NS
LANES = 16


def _route_body(xf_ref, Wr_ref, br_ref, rank_ref, wts_ref, te_ref, oh_ref):
    xf = xf_ref[...]
    logits = jnp.dot(xf, Wr_ref[...], preferred_element_type=jnp.float32) + br_ref[...]
    probs = jax.nn.softmax(logits, axis=-1)
    cols = lax.broadcasted_iota(jnp.int32, (T, E), 1)
    p1 = jnp.max(probs, axis=-1)
    a1 = jnp.min(jnp.where(probs == p1[:, None], cols, E), axis=-1)
    masked = jnp.where(cols == a1[:, None], -jnp.inf, probs)
    p2 = jnp.max(masked, axis=-1)
    a2 = jnp.min(jnp.where(masked == p2[:, None], cols, E), axis=-1)
    denom = p1 + p2
    w_all = jnp.concatenate([p1 / denom, p2 / denom], 0)
    wts_ref[...] = jnp.broadcast_to(w_all[:, None], (P, LANES))

    e_all = jnp.concatenate([a1, a2], 0)
    ecols = lax.broadcasted_iota(jnp.int32, (P, E), 1)
    oh = (e_all[:, None] == ecols).astype(jnp.float32)
    oh_ref[...] = oh.reshape(NBLK, BLK, E)
    blocksums = jnp.sum(oh_ref[...], axis=1)
    total = jnp.sum(blocksums, axis=0).astype(jnp.int32)
    pc = (((total + BLK - 1) // BLK) * BLK).astype(jnp.float32)
    r8 = lax.broadcasted_iota(jnp.int32, (E, E), 0)
    c8 = lax.broadcasted_iota(jnp.int32, (E, E), 1)
    tri8 = (r8 >= c8).astype(jnp.float32)
    cumpc = jnp.dot(tri8, pc[:, None], preferred_element_type=jnp.float32)
    poff = (cumpc - pc[:, None])[:, 0]

    ti = lax.broadcasted_iota(jnp.int32, (BLK, E), 0) * BLK
    cumpc_i = cumpc[:, 0].astype(jnp.int32)
    te = jnp.sum((ti >= cumpc_i[None, :]).astype(jnp.int32), axis=1)
    te_ref[0, :] = jnp.minimum(te, E - 1)

    rr = lax.broadcasted_iota(jnp.int32, (BLK, BLK), 0)
    cc = lax.broadcasted_iota(jnp.int32, (BLK, BLK), 1)
    tri128 = (rr > cc).astype(jnp.float32)

    def body(b, run):
        oh_b = oh_ref[b]
        within = jnp.dot(tri128, oh_b, preferred_element_type=jnp.float32)
        rank_b = jnp.sum(oh_b * (poff[None, :] + run + within), axis=1)
        rank_ref[b, :] = rank_b.astype(jnp.int32)
        return run + jnp.sum(oh_b, axis=0, keepdims=True)

    lax.fori_loop(0, NBLK, body, jnp.zeros((1, E), jnp.float32))


def _route(xf, Wr, br2):
    return pl.pallas_call(
        _route_body,
        out_shape=[
            jax.ShapeDtypeStruct((NBLK, BLK), jnp.int32),
            jax.ShapeDtypeStruct((P, LANES), jnp.float32),
            jax.ShapeDtypeStruct((1, BLK), jnp.int32),
        ],
        scratch_shapes=[pltpu.VMEM((NBLK, BLK, E), jnp.float32)],
    )(xf, Wr, br2)


_sc_mesh = plsc.VectorSubcoreMesh(
    core_axis_name="c", subcore_axis_name="s", num_cores=NC, num_subcores=NS)

_PAIRS_PER_W = P // NW
_DCH = 128


@functools.partial(
    pl.kernel,
    out_type=jax.ShapeDtypeStruct((XS, D), jnp.float32),
    mesh=_sc_mesh,
    scratch_types=[
        pltpu.VMEM((_DCH,), jnp.int32),
        pltpu.VMEM((_DCH, D), jnp.float32),
        pltpu.SemaphoreType.DMA,
    ],
)
def _dispatch(xf_hbm, rank_hbm, xs_hbm, idx_v, rows_v, sem):
    wid = lax.axis_index("s") * NC + lax.axis_index("c")
    for ch in range(_PAIRS_PER_W // _DCH):
        pbase = wid * _PAIRS_PER_W + ch * _DCH
        tbase = lax.rem(pbase, T)
        pltpu.sync_copy(rank_hbm.at[pl.ds(pbase, _DCH)], idx_v)
        pltpu.sync_copy(xf_hbm.at[pl.ds(tbase, _DCH)], rows_v)
        pltpu.async_copy(rows_v, xs_hbm.at[idx_v], sem).wait()


def _glu_body(te_ref, xs_ref, W1_ref, W3_ref, W2_ref, out_ref):
    xs = xs_ref[...]
    h1 = jnp.dot(xs, W1_ref[0], preferred_element_type=jnp.float32)
    h3 = jnp.dot(xs, W3_ref[0], preferred_element_type=jnp.float32)
    out_ref[...] = jnp.dot(jax.nn.silu(h1) * h3, W2_ref[0],
                           preferred_element_type=jnp.float32)


def _glu(te, xs, W1, W3, W2):
    grid_spec = pltpu.PrefetchScalarGridSpec(
        num_scalar_prefetch=1,
        grid=(NT,),
        in_specs=[
            pl.BlockSpec((BLK, D), lambda i, te: (i, 0)),
            pl.BlockSpec((1, D, H), lambda i, te: (te[i], 0, 0)),
            pl.BlockSpec((1, D, H), lambda i, te: (te[i], 0, 0)),
            pl.BlockSpec((1, H, D), lambda i, te: (te[i], 0, 0)),
        ],
        out_specs=pl.BlockSpec((BLK, D), lambda i, te: (i, 0)),
    )
    return pl.pallas_call(
        _glu_body,
        grid_spec=grid_spec,
        out_shape=jax.ShapeDtypeStruct((XS, D), jnp.float32),
        compiler_params=pltpu.CompilerParams(
            dimension_semantics=("arbitrary",)),
    )(te, xs, W1, W3, W2)


_TOK_PER_W = T // NW
_CCH = 64


@functools.partial(
    pl.kernel,
    out_type=jax.ShapeDtypeStruct((T, D), jnp.float32),
    mesh=_sc_mesh,
    scratch_types=[
        pltpu.VMEM((_CCH,), jnp.int32),
        pltpu.VMEM((_CCH,), jnp.int32),
        pltpu.VMEM((_CCH, LANES), jnp.float32),
        pltpu.VMEM((_CCH, LANES), jnp.float32),
        pltpu.VMEM((_CCH, D), jnp.float32),
        pltpu.VMEM((_CCH, D), jnp.float32),
        pltpu.SemaphoreType.DMA,
    ],
)
def _combine(os_hbm, rank_hbm, wts_hbm, out_hbm,
             i0_v, i1_v, w0_v, w1_v, ra_v, rb_v, sem):
    wid = lax.axis_index("s") * NC + lax.axis_index("c")
    for ch in range(_TOK_PER_W // _CCH):
        tb = wid * _TOK_PER_W + ch * _CCH
        pltpu.sync_copy(rank_hbm.at[pl.ds(tb, _CCH)], i0_v)
        pltpu.sync_copy(rank_hbm.at[pl.ds(T + tb, _CCH)], i1_v)
        pltpu.sync_copy(wts_hbm.at[pl.ds(tb, _CCH)], w0_v)
        pltpu.sync_copy(wts_hbm.at[pl.ds(T + tb, _CCH)], w1_v)
        pltpu.async_copy(os_hbm.at[i0_v], ra_v, sem).wait()
        pltpu.async_copy(os_hbm.at[i1_v], rb_v, sem).wait()

        def row_body(j, carry):
            w0 = w0_v[j, :]
            w1 = w1_v[j, :]
            for i in range(D // LANES):
                sl = pl.ds(i * LANES, LANES)
                ra_v[j, sl] = ra_v[j, sl] * w0 + rb_v[j, sl] * w1
            return carry

        lax.fori_loop(0, _CCH, row_body, 0)
        pltpu.sync_copy(ra_v, out_hbm.at[pl.ds(tb, _CCH)])


@jax.jit
def _moe(xf, Wr, br2, W1, W3, W2):
    rank2d, wts, te2d = _route(xf, Wr, br2)
    rank = rank2d.reshape(P)
    te = te2d.reshape(BLK)[:NT]
    xs = _dispatch(xf, rank)
    os_ = _glu(te, xs, W1, W3, W2)
    return _combine(os_, rank, wts)


def kernel(x, Wr, br, W1, W3, W2):
    b, s, d = x.shape
    xf = x.reshape(b * s, d)
    out = _moe(xf, Wr, br.reshape(1, E), W1, W3, W2)
    return out.reshape(b, s, d)


# speedup vs baseline: 1.1340x; 1.1340x over previous
import jax
import jax.numpy as jnp
from jax.experimental import pallas as pl
from jax.experimental.pallas import tpu as pltpu

B, S, D, H, E, TOPK = 2, 2048, 768, 1024, 8, 2
T = B * S
TT = 2048


def _moe_body(xf_ref, Wr_ref, br_ref, W1_ref, W3_ref, W2_ref, out_ref):
    e = pl.program_id(1)
    xf = xf_ref[...]
    logits = jnp.dot(xf, Wr_ref[...], preferred_element_type=jnp.float32)
    logits = logits + br_ref[...]
    probs = jax.nn.softmax(logits, axis=-1)
    cols = jax.lax.broadcasted_iota(jnp.int32, probs.shape, 1)
    a1 = jnp.argmax(probs, axis=-1)
    p1 = jnp.max(probs, axis=-1)
    masked = jnp.where(cols == a1[:, None], -jnp.inf, probs)
    a2 = jnp.argmax(masked, axis=-1)
    p2 = jnp.max(masked, axis=-1)
    denom = p1 + p2
    we = (p1 * (a1 == e) + p2 * (a2 == e)) / denom

    xb = xf.astype(jnp.bfloat16)
    w1 = W1_ref[0].astype(jnp.bfloat16)
    w3 = W3_ref[0].astype(jnp.bfloat16)
    w2 = W2_ref[0].astype(jnp.bfloat16)
    h1 = jnp.dot(xb, w1, preferred_element_type=jnp.float32)
    h3 = jnp.dot(xb, w3, preferred_element_type=jnp.float32)
    act = (jax.nn.silu(h1) * h3).astype(jnp.bfloat16)
    y = jnp.dot(act, w2, preferred_element_type=jnp.float32)
    contrib = we[:, None] * y

    @pl.when(e == 0)
    def _():
        out_ref[...] = contrib

    @pl.when(e != 0)
    def _():
        out_ref[...] += contrib


@jax.jit
def _moe(xf, Wr, br2, W1, W3, W2):
    n_tt = T // TT
    return pl.pallas_call(
        _moe_body,
        grid=(n_tt, E),
        in_specs=[
            pl.BlockSpec((TT, D), lambda t, e: (t, 0)),
            pl.BlockSpec((D, E), lambda t, e: (0, 0)),
            pl.BlockSpec((1, E), lambda t, e: (0, 0)),
            pl.BlockSpec((1, D, H), lambda t, e: (e, 0, 0)),
            pl.BlockSpec((1, D, H), lambda t, e: (e, 0, 0)),
            pl.BlockSpec((1, H, D), lambda t, e: (e, 0, 0)),
        ],
        out_specs=pl.BlockSpec((TT, D), lambda t, e: (t, 0)),
        out_shape=jax.ShapeDtypeStruct((T, D), jnp.float32),
        compiler_params=pltpu.CompilerParams(
            dimension_semantics=("arbitrary", "arbitrary"),
        ),
    )(xf, Wr, br2, W1, W3, W2)


def kernel(x, Wr, br, W1, W3, W2):
    b, s, d = x.shape
    xf = x.reshape(b * s, d)
    out = _moe(xf, Wr, br.reshape(1, E), W1, W3, W2)
    return out.reshape(b, s, d)
